# baseline (speedup 1.0000x reference)
import functools

import jax
import jax.numpy as jnp
from jax import lax
from jax.experimental import pallas as pl
from jax.experimental.pallas import tpu as pltpu
from jax.experimental.pallas import tpu_sc as plsc


def _shuffle_block(buf_in, buf_out, nrows, r0=0):
    c0v = lax.broadcasted_iota(jnp.int32, (16,), 0)
    c1v = c0v + 16
    for r in range(r0, r0 + nrows):
        rv = jnp.full((16,), r, jnp.int32)
        o = (r - r0) * 32
        buf_out[pl.ds(o, 16)] = plsc.load_gather(buf_in, [c0v, rv])
        buf_out[pl.ds(o + 16, 16)] = plsc.load_gather(buf_in, [c1v, rv])


def _make_transpose(v, d):
    info = plsc.get_sparse_core_info()
    nc, ns = info.num_cores, info.num_subcores
    nw = nc * ns
    assert d == 32
    nfull = v // 128
    tail = v % 128
    per_w = nfull // nw
    extra = nfull % nw
    assert per_w % 2 == 0

    mesh = plsc.VectorSubcoreMesh(core_axis_name="c", subcore_axis_name="s")

    @functools.partial(
        pl.kernel,
        mesh=mesh,
        compiler_params=pltpu.CompilerParams(needs_layout_passes=False),
        out_type=jax.ShapeDtypeStruct((v * d,), jnp.float32),
        scratch_types=[
            pltpu.VMEM((d, 128), jnp.float32),
            pltpu.VMEM((d, 128), jnp.float32),
            pltpu.VMEM((128 * d,), jnp.float32),
            pltpu.VMEM((128 * d,), jnp.float32),
            pltpu.VMEM((d, tail if tail else 1), jnp.float32),
            pltpu.SemaphoreType.DMA,
            pltpu.SemaphoreType.DMA,
            pltpu.SemaphoreType.DMA,
            pltpu.SemaphoreType.DMA,
        ],
    )
    def transpose_kernel(tt_hbm, tp_hbm, bin0, bin1, bout0, bout1, btail,
                         isem0, isem1, osem0, osem1):
        wid = lax.axis_index("s") * nc + lax.axis_index("c")
        bins = [bin0, bin1]
        bouts = [bout0, bout1]
        isems = [isem0, isem1]
        osems = [osem0, osem1]

        def in_slice(blk):
            return tt_hbm.at[:, pl.ds(pl.multiple_of(blk * 128, 128), 128)]

        def out_slice(blk):
            return tp_hbm.at[pl.ds(pl.multiple_of(blk * (128 * d), 8),
                                   128 * d)]

        def start_in(blk, p):
            pltpu.async_copy(in_slice(blk), bins[p], isems[p])

        def wait_in(p):
            pltpu.make_async_copy(in_slice(0), bins[p], isems[p]).wait()

        def start_out(blk, p):
            pltpu.async_copy(bouts[p], out_slice(blk), osems[p])

        def wait_out(p):
            pltpu.make_async_copy(bouts[p], out_slice(0), osems[p]).wait()

        start_in(wid, 0)
        for j in (0, 1):
            wait_in(j)
            nblk = wid + nw * (j + 1)

            @pl.when(nblk < nfull)
            def _():
                start_in(nblk, 1 - j)

            _shuffle_block(bins[j], bouts[j], 128)
            start_out(wid + nw * j, j)

        def body(i2, carry):
            for k in (0, 1):
                j = i2 * 2 + k
                wait_in(k)
                nblk = wid + nw * (j + 1)

                @pl.when(nblk < nfull)
                def _():
                    start_in(nblk, 1 - k)

                wait_out(k)
                _shuffle_block(bins[k], bouts[k], 128)
                start_out(wid + nw * j, k)
            return carry

        lax.fori_loop(1, per_w // 2, body, 0)
        wait_out(0)
        wait_out(1)

        if extra:
            @pl.when(wid < extra)
            def _():
                wait_in(0)
                _shuffle_block(bins[0], bouts[0], 128)
                pltpu.sync_copy(bouts[0], out_slice(wid + nw * per_w))

        if tail:
            @pl.when(wid == extra)
            def _():
                pltpu.sync_copy(tt_hbm.at[:, pl.ds(nfull * 128, tail)], btail)
                _shuffle_block(btail, bouts[0], tail)
                pltpu.sync_copy(
                    bouts[0].at[pl.ds(0, tail * d)],
                    tp_hbm.at[pl.ds(nfull * (128 * d), tail * d)])

    return transpose_kernel


def _make_gather(n, v, d):
    info = plsc.get_sparse_core_info()
    nc, ns = info.num_cores, info.num_subcores
    nw = nc * ns
    assert n % nw == 0
    b_per_w = n // nw
    chunk = 1664
    while b_per_w % chunk != 0:
        chunk //= 2
    nchunks = b_per_w // chunk

    mesh = plsc.VectorSubcoreMesh(core_axis_name="c", subcore_axis_name="s")

    @functools.partial(
        pl.kernel,
        mesh=mesh,
        compiler_params=pltpu.CompilerParams(use_tc_tiling_on_sc=False),
        out_type=jax.ShapeDtypeStruct((n, d), jnp.float32),
        scratch_types=[
            pltpu.VMEM((b_per_w,), jnp.int32),
            pltpu.VMEM((chunk, d), jnp.float32),
            pltpu.VMEM((chunk, d), jnp.float32),
            pltpu.SemaphoreType.DMA,
            pltpu.SemaphoreType.DMA,
            pltpu.SemaphoreType.DMA,
            pltpu.SemaphoreType.DMA,
        ],
    )
    def gather_kernel(table_hbm, idx_hbm, out_hbm, idx_v,
                      rows0, rows1, gsem0, gsem1, osem0, osem1):
        wid = lax.axis_index("s") * nc + lax.axis_index("c")
        base = wid * b_per_w
        pltpu.sync_copy(idx_hbm.at[pl.ds(base, b_per_w)], idx_v)

        rows = [rows0, rows1]
        gsems = [gsem0, gsem1]
        osems = [osem0, osem1]
        g_desc = [None, None]
        o_desc = [None, None]

        def issue_gather(g):
            bb = g % 2
            g_desc[bb] = pltpu.async_copy(
                table_hbm.at[idx_v.at[pl.ds(g * chunk, chunk)]],
                rows[bb], gsems[bb])

        def issue_out(g):
            bb = g % 2
            o_desc[bb] = pltpu.async_copy(
                rows[bb], out_hbm.at[pl.ds(base + g * chunk, chunk)],
                osems[bb])

        issue_gather(0)
        for g in range(nchunks):
            bb = g % 2
            g_desc[bb].wait()
            if g >= 1:
                o_desc[1 - bb].wait()
            if g + 1 < nchunks:
                issue_gather(g + 1)
            issue_out(g)
        o_desc[(nchunks - 1) % 2].wait()

    return gather_kernel


def kernel(inputs, table):
    b, f = inputs.shape
    v, d = table.shape
    n = b * f
    flat_idx = inputs.reshape(n).astype(jnp.int32)
    tp = _make_transpose(v, d)(table.T).reshape(v, d)
    out = _make_gather(n, v, d)(tp, flat_idx)
    return out.reshape(b, f, d)


# speedup vs baseline: 1.9889x; 1.9889x over previous
import functools

import jax
import jax.numpy as jnp
from jax import lax
from jax.experimental import pallas as pl
from jax.experimental.pallas import tpu as pltpu
from jax.experimental.pallas import tpu_sc as plsc


def _shuffle_block(buf_in, buf_out, nrows):
    iota = lax.broadcasted_iota(jnp.int32, (16,), 0)

    def kbody(k, carry):
        perm = jnp.bitwise_and(iota + k, 15)
        st0 = perm * 32 + iota
        for rb in range(0, nrows, 16):
            for c0 in (0, 16):
                v = plsc.load_gather(buf_in, [iota + c0, perm + rb])
                plsc.store_scatter(buf_out, [st0 + (rb * 32 + c0)], v)
        return carry

    lax.fori_loop(0, 16, kbody, 0)


def _make_transpose(v, d):
    info = plsc.get_sparse_core_info()
    nc, ns = info.num_cores, info.num_subcores
    nw = nc * ns
    assert d == 32
    nfull = v // 128
    tail = v % 128
    per_w = nfull // nw
    extra = nfull % nw
    assert per_w % 2 == 0

    mesh = plsc.VectorSubcoreMesh(core_axis_name="c", subcore_axis_name="s")

    @functools.partial(
        pl.kernel,
        mesh=mesh,
        compiler_params=pltpu.CompilerParams(needs_layout_passes=False),
        out_type=jax.ShapeDtypeStruct((v * d,), jnp.float32),
        scratch_types=[
            pltpu.VMEM((d, 128), jnp.float32),
            pltpu.VMEM((d, 128), jnp.float32),
            pltpu.VMEM((128 * d,), jnp.float32),
            pltpu.VMEM((128 * d,), jnp.float32),
            pltpu.VMEM((d, tail if tail else 1), jnp.float32),
            pltpu.SemaphoreType.DMA,
            pltpu.SemaphoreType.DMA,
            pltpu.SemaphoreType.DMA,
            pltpu.SemaphoreType.DMA,
        ],
    )
    def transpose_kernel(tt_hbm, tp_hbm, bin0, bin1, bout0, bout1, btail,
                         isem0, isem1, osem0, osem1):
        wid = lax.axis_index("s") * nc + lax.axis_index("c")
        bins = [bin0, bin1]
        bouts = [bout0, bout1]
        isems = [isem0, isem1]
        osems = [osem0, osem1]

        def in_slice(blk):
            return tt_hbm.at[:, pl.ds(pl.multiple_of(blk * 128, 128), 128)]

        def out_slice(blk):
            return tp_hbm.at[pl.ds(pl.multiple_of(blk * (128 * d), 8),
                                   128 * d)]

        def start_in(blk, p):
            pltpu.async_copy(in_slice(blk), bins[p], isems[p])

        def wait_in(p):
            pltpu.make_async_copy(in_slice(0), bins[p], isems[p]).wait()

        def start_out(blk, p):
            pltpu.async_copy(bouts[p], out_slice(blk), osems[p])

        def wait_out(p):
            pltpu.make_async_copy(bouts[p], out_slice(0), osems[p]).wait()

        start_in(wid, 0)
        for j in (0, 1):
            wait_in(j)
            nblk = wid + nw * (j + 1)

            @pl.when(nblk < nfull)
            def _():
                start_in(nblk, 1 - j)

            _shuffle_block(bins[j], bouts[j], 128)
            start_out(wid + nw * j, j)

        def body(i2, carry):
            for k in (0, 1):
                j = i2 * 2 + k
                wait_in(k)
                nblk = wid + nw * (j + 1)

                @pl.when(nblk < nfull)
                def _():
                    start_in(nblk, 1 - k)

                wait_out(k)
                _shuffle_block(bins[k], bouts[k], 128)
                start_out(wid + nw * j, k)
            return carry

        lax.fori_loop(1, per_w // 2, body, 0)
        wait_out(0)
        wait_out(1)

        if extra:
            @pl.when(wid < extra)
            def _():
                wait_in(0)
                _shuffle_block(bins[0], bouts[0], 128)
                pltpu.sync_copy(bouts[0], out_slice(wid + nw * per_w))

        if tail:
            @pl.when(wid == extra)
            def _():
                pltpu.sync_copy(tt_hbm.at[:, pl.ds(nfull * 128, tail)], btail)
                _shuffle_block(btail, bouts[0], tail)
                pltpu.sync_copy(
                    bouts[0].at[pl.ds(0, tail * d)],
                    tp_hbm.at[pl.ds(nfull * (128 * d), tail * d)])

    return transpose_kernel


def _make_gather(n, v, d):
    info = plsc.get_sparse_core_info()
    nc, ns = info.num_cores, info.num_subcores
    nw = nc * ns
    assert n % nw == 0
    b_per_w = n // nw
    chunk = 1664
    while b_per_w % chunk != 0:
        chunk //= 2
    nchunks = b_per_w // chunk

    mesh = plsc.VectorSubcoreMesh(core_axis_name="c", subcore_axis_name="s")

    @functools.partial(
        pl.kernel,
        mesh=mesh,
        compiler_params=pltpu.CompilerParams(use_tc_tiling_on_sc=False),
        out_type=jax.ShapeDtypeStruct((n, d), jnp.float32),
        scratch_types=[
            pltpu.VMEM((b_per_w,), jnp.int32),
            pltpu.VMEM((chunk, d), jnp.float32),
            pltpu.VMEM((chunk, d), jnp.float32),
            pltpu.SemaphoreType.DMA,
            pltpu.SemaphoreType.DMA,
            pltpu.SemaphoreType.DMA,
            pltpu.SemaphoreType.DMA,
        ],
    )
    def gather_kernel(table_hbm, idx_hbm, out_hbm, idx_v,
                      rows0, rows1, gsem0, gsem1, osem0, osem1):
        wid = lax.axis_index("s") * nc + lax.axis_index("c")
        base = wid * b_per_w
        pltpu.sync_copy(idx_hbm.at[pl.ds(base, b_per_w)], idx_v)

        rows = [rows0, rows1]
        gsems = [gsem0, gsem1]
        osems = [osem0, osem1]
        g_desc = [None, None]
        o_desc = [None, None]

        def issue_gather(g):
            bb = g % 2
            g_desc[bb] = pltpu.async_copy(
                table_hbm.at[idx_v.at[pl.ds(g * chunk, chunk)]],
                rows[bb], gsems[bb])

        def issue_out(g):
            bb = g % 2
            o_desc[bb] = pltpu.async_copy(
                rows[bb], out_hbm.at[pl.ds(base + g * chunk, chunk)],
                osems[bb])

        issue_gather(0)
        for g in range(nchunks):
            bb = g % 2
            g_desc[bb].wait()
            if g >= 1:
                o_desc[1 - bb].wait()
            if g + 1 < nchunks:
                issue_gather(g + 1)
            issue_out(g)
        o_desc[(nchunks - 1) % 2].wait()

    return gather_kernel


def kernel(inputs, table):
    b, f = inputs.shape
    v, d = table.shape
    n = b * f
    flat_idx = inputs.reshape(n).astype(jnp.int32)
    tp = _make_transpose(v, d)(table.T).reshape(v, d)
    out = _make_gather(n, v, d)(tp, flat_idx)
    return out.reshape(b, f, d)
